# baseline (speedup 1.0000x reference)
import functools

import jax
import jax.numpy as jnp
from jax import lax
from jax.experimental import pallas as pl
from jax.experimental.pallas import tpu as pltpu
from jax.experimental.pallas import tpu_sc as plsc

N_NODES = 10000
N_PAD = 10240
D_IN = 128
D_MID = 64
N_EDGES = 320000

NC = 2
NS = 16
NW = NC * NS
CHUNK = 128
CPT = 80
E_PAD = NW * CPT * CHUNK
RPT = N_PAD // NS

_MESH = plsc.VectorSubcoreMesh(core_axis_name="c", subcore_axis_name="s")
_SC_PARAMS = pltpu.CompilerParams(use_tc_tiling_on_sc=False)


def _fill(ref, val, n):
    v = jnp.full((16,), val, jnp.float32)
    for k in range(n // 16):
        ref[pl.ds(k * 16, 16)] = v


DSLAB = 512
DSPT = E_PAD // NW // DSLAB


@functools.partial(
    pl.kernel,
    out_type=jax.ShapeDtypeStruct((NC, N_PAD), jnp.float32),
    mesh=_MESH,
    scratch_types=[
        pltpu.VMEM((DSPT, DSLAB), jnp.int32),
        pltpu.VMEM((DSLAB,), jnp.float32),
        pltpu.VMEM((DSLAB,), jnp.float32),
        pltpu.VMEM_SHARED((N_PAD,), jnp.float32),
        pltpu.SemaphoreType.DMA,
    ],
    compiler_params=_SC_PARAMS,
)
def _deg_kernel(dst_hbm, out_hbm, dst_v, ones_v, zeros_v, acc, sem):
    c = lax.axis_index("c")
    s = lax.axis_index("s")
    wid = c * NS + s
    pltpu.sync_copy(dst_hbm.at[pl.ds(wid * DSPT, DSPT)], dst_v)
    _fill(ones_v, 1.0, DSLAB)
    _fill(zeros_v, 0.0, DSLAB)
    base = s * RPT
    off = 0
    while off < RPT:
        step = min(DSLAB, RPT - off)
        pltpu.sync_copy(zeros_v.at[pl.ds(0, step)],
                        acc.at[pl.ds(base + off, step)])
        off += step
    plsc.subcore_barrier()

    def group(i, _):
        for b in range(4):
            pltpu.async_copy(ones_v, acc.at[dst_v.at[i * 4 + b]], sem,
                             add=True)
        for b in range(4):
            pltpu.make_async_copy(ones_v, acc.at[dst_v.at[i * 4 + b]],
                                  sem).wait()
        return 0

    lax.fori_loop(0, DSPT // 4, group, 0)
    plsc.subcore_barrier()
    pltpu.sync_copy(acc.at[pl.ds(base, RPT)], out_hbm.at[c, pl.ds(base, RPT)])


SLAB = 128


def _make_agg(core_split):
    ept = (E_PAD // NS if core_split else E_PAD // NW)
    ns = ept // SLAB
    D = D_MID

    @functools.partial(
        pl.kernel,
        out_type=jax.ShapeDtypeStruct((NC, N_PAD, D), jnp.float32),
        mesh=_MESH,
        scratch_types=[
            pltpu.VMEM((ns, SLAB), jnp.int32),
            pltpu.VMEM((ns, SLAB), jnp.int32),
            pltpu.VMEM((SLAB, D), jnp.float32),
            pltpu.VMEM((SLAB, D), jnp.float32),
            pltpu.VMEM((SLAB, D), jnp.float32),
            pltpu.VMEM((SLAB, D), jnp.float32),
            pltpu.VMEM_SHARED((N_PAD, D), jnp.float32),
            pltpu.SemaphoreType.DMA,
            pltpu.SemaphoreType.DMA,
            pltpu.SemaphoreType.DMA,
            pltpu.SemaphoreType.DMA,
        ],
        compiler_params=_SC_PARAMS,
    )
    def agg(xs_hbm, src_hbm, dst_hbm, out_hbm, src_v, dst_v, rows0, rows1,
            rows2, rows3, acc, gsem0, gsem1, ssem0, ssem1):
        c = lax.axis_index("c")
        s = lax.axis_index("s")
        if core_split:
            sbase = s * ns
            xsc = xs_hbm.at[c]
        else:
            sbase = (c * NS + s) * ns
            xsc = xs_hbm
        pltpu.sync_copy(src_hbm.at[pl.ds(sbase, ns)], src_v)
        pltpu.sync_copy(dst_hbm.at[pl.ds(sbase, ns)], dst_v)

        def zrow(r, _):
            for k in range(D // 16):
                rows0[r, pl.ds(k * 16, 16)] = jnp.zeros((16,), jnp.float32)
            return 0

        lax.fori_loop(0, SLAB, zrow, 0)
        base = s * RPT
        off = 0
        while off < RPT:
            step = min(SLAB, RPT - off)
            pltpu.sync_copy(rows0.at[pl.ds(0, step)],
                            acc.at[pl.ds(base + off, step)])
            off += step
        plsc.subcore_barrier()

        def sidx(j):
            return src_v.at[j]

        def didx(j):
            return dst_v.at[j]

        rows = [rows0, rows1, rows2, rows3]
        gsem = [gsem0, gsem1]
        ssem = [ssem0, ssem1]

        pltpu.async_copy(xsc.at[sidx(0)], rows[0], gsem[0])
        pltpu.async_copy(xsc.at[sidx(1)], rows[1], gsem[1])

        def quad(i, _):
            for b in range(4):
                j = i * 4 + b
                pltpu.make_async_copy(
                    xsc.at[sidx(j)], rows[b], gsem[b % 2]).wait()
                pltpu.async_copy(rows[b], acc.at[didx(j)], ssem[b % 2],
                                 add=True)

                @pl.when(j >= 2)
                def _():
                    pltpu.make_async_copy(
                        rows[(b + 2) % 4], acc.at[didx(j - 2)],
                        ssem[b % 2]).wait()

                @pl.when(j + 2 < ns)
                def _():
                    pltpu.async_copy(
                        xsc.at[sidx(j + 2)], rows[(b + 2) % 4], gsem[b % 2])
            return 0

        lax.fori_loop(0, ns // 4, quad, 0)
        pltpu.make_async_copy(
            rows[(ns - 2) % 4], acc.at[didx(ns - 2)], ssem[0]).wait()
        pltpu.make_async_copy(
            rows[(ns - 1) % 4], acc.at[didx(ns - 1)], ssem[1]).wait()

        plsc.subcore_barrier()
        pltpu.sync_copy(acc.at[pl.ds(base, RPT)],
                        out_hbm.at[c, pl.ds(base, RPT)])

    return agg


_agg_mid = _make_agg(core_split=False)


_NS2 = (E_PAD // NS) // SLAB


@functools.partial(
    pl.kernel,
    out_type=jax.ShapeDtypeStruct((N_NODES, D_IN), jnp.float32),
    mesh=_MESH,
    scratch_types=[
        pltpu.VMEM((_NS2, SLAB), jnp.int32),
        pltpu.VMEM((_NS2, SLAB), jnp.int32),
        pltpu.VMEM((SLAB, D_MID), jnp.float32),
        pltpu.VMEM((SLAB, D_MID), jnp.float32),
        pltpu.VMEM((SLAB, D_MID), jnp.float32),
        pltpu.VMEM((SLAB, D_MID), jnp.float32),
        pltpu.VMEM((RPT,), jnp.float32),
        pltpu.VMEM((D_MID,), jnp.float32),
        pltpu.VMEM_SHARED((N_PAD, D_MID), jnp.float32),
        pltpu.SemaphoreType.DMA,
        pltpu.SemaphoreType.DMA,
        pltpu.SemaphoreType.DMA,
        pltpu.SemaphoreType.DMA,
    ],
    compiler_params=_SC_PARAMS,
)
def _agg_final(xs_hbm, src_hbm, dst_hbm, dis_hbm, b2_hbm, out_hbm,
               src_v, dst_v, rows0, rows1, rows2, rows3, dis_v, b2_v,
               acc, gsem0, gsem1, ssem0, ssem1):
    ns = _NS2
    D = D_MID
    c = lax.axis_index("c")
    s = lax.axis_index("s")
    sbase = s * ns
    xsc = xs_hbm.at[c]
    pltpu.sync_copy(src_hbm.at[pl.ds(sbase, ns)], src_v)
    pltpu.sync_copy(dst_hbm.at[pl.ds(sbase, ns)], dst_v)
    base = s * RPT
    pltpu.sync_copy(dis_hbm.at[pl.ds(base, RPT)], dis_v)
    pltpu.sync_copy(b2_hbm.at[c], b2_v)

    def zrow(r, _):
        for k in range(D // 16):
            rows0[r, pl.ds(k * 16, 16)] = jnp.zeros((16,), jnp.float32)
        return 0

    lax.fori_loop(0, SLAB, zrow, 0)
    off = 0
    while off < RPT:
        step = min(SLAB, RPT - off)
        pltpu.sync_copy(rows0.at[pl.ds(0, step)],
                        acc.at[pl.ds(base + off, step)])
        off += step
    plsc.subcore_barrier()

    rows = [rows0, rows1, rows2, rows3]
    gsem = [gsem0, gsem1]
    ssem = [ssem0, ssem1]

    pltpu.async_copy(xsc.at[src_v.at[0]], rows[0], gsem[0])
    pltpu.async_copy(xsc.at[src_v.at[1]], rows[1], gsem[1])

    def quad(i, _):
        for b in range(4):
            j = i * 4 + b
            pltpu.make_async_copy(
                xsc.at[src_v.at[j]], rows[b], gsem[b % 2]).wait()
            pltpu.async_copy(rows[b], acc.at[dst_v.at[j]], ssem[b % 2],
                             add=True)

            @pl.when(j >= 2)
            def _():
                pltpu.make_async_copy(
                    rows[(b + 2) % 4], acc.at[dst_v.at[j - 2]],
                    ssem[b % 2]).wait()

            @pl.when(j + 2 < ns)
            def _():
                pltpu.async_copy(
                    xsc.at[src_v.at[j + 2]], rows[(b + 2) % 4], gsem[b % 2])
        return 0

    lax.fori_loop(0, ns // 4, quad, 0)
    pltpu.make_async_copy(
        rows[(ns - 2) % 4], acc.at[dst_v.at[ns - 2]], ssem[0]).wait()
    pltpu.make_async_copy(
        rows[(ns - 1) % 4], acc.at[dst_v.at[ns - 1]], ssem[1]).wait()

    plsc.subcore_barrier()

    for t in range(RPT // SLAB):
        r0 = base + t * SLAB
        pltpu.sync_copy(acc.at[pl.ds(r0, SLAB)], rows0)
        pltpu.sync_copy(xsc.at[pl.ds(r0, SLAB)], rows1)

        def fgroup(g, _):
            dvec = dis_v[pl.ds(t * SLAB + g * 16, 16)]
            for rr in range(16):
                r = g * 16 + rr
                dv = dvec[rr]
                for k in range(D // 16):
                    sl = pl.ds(k * 16, 16)
                    rows0[r, sl] = ((rows0[r, sl] + rows1[r, sl]) * dv
                                    + b2_v[sl])
            return 0

        lax.fori_loop(0, SLAB // 16, fgroup, 0)
        tail = N_NODES % SLAB

        @pl.when(r0 + SLAB <= N_NODES)
        def _():
            pltpu.sync_copy(rows0,
                            out_hbm.at[pl.ds(r0, SLAB), pl.ds(c * D, D)])

        @pl.when(jnp.logical_and(r0 < N_NODES, r0 + SLAB > N_NODES))
        def _():
            pltpu.sync_copy(
                rows0.at[pl.ds(0, tail)],
                out_hbm.at[pl.ds(r0, tail), pl.ds(c * D, D)])


def _tc1_body(deg_ref, x_ref, w_ref, xs_ref, dis_ref):
    deg = deg_ref[0] + deg_ref[1] + 1.0
    dis1 = lax.rsqrt(deg)
    dis_ref[...] = dis1
    dis = dis1.reshape(N_PAD, 1)
    xw = jnp.dot(x_ref[...], w_ref[...], preferred_element_type=jnp.float32)
    xs_ref[pl.ds(0, N_NODES), :] = xw * dis[:N_NODES]
    xs_ref[pl.ds(N_NODES, N_PAD - N_NODES), :] = jnp.zeros(
        (N_PAD - N_NODES, D_MID), jnp.float32)


def _tc2_body(agg_ref, xs_ref, dis_ref, b_ref, w_ref, xs2_ref):
    dis = dis_ref[...].reshape(N_PAD, 1)
    h = jnp.maximum(
        dis * (agg_ref[0] + agg_ref[1] + xs_ref[...]) + b_ref[...], 0.0)
    xw2 = jnp.dot(h, w_ref[...], preferred_element_type=jnp.float32)
    xs2 = xw2 * dis
    xs2_ref[0] = xs2[:, :D_MID]
    xs2_ref[1] = xs2[:, D_MID:]


_tc1 = pl.pallas_call(
    _tc1_body,
    out_shape=[
        jax.ShapeDtypeStruct((N_PAD, D_MID), jnp.float32),
        jax.ShapeDtypeStruct((N_PAD,), jnp.float32),
    ],
)

_tc2 = pl.pallas_call(
    _tc2_body,
    out_shape=jax.ShapeDtypeStruct((NC, N_PAD, D_MID), jnp.float32),
)


def kernel(x, edge_index, W1, b1, W2, b2):
    src = edge_index[0].astype(jnp.int32)
    dst = edge_index[1].astype(jnp.int32)
    pad = E_PAD - N_EDGES
    ar = jnp.arange(pad, dtype=jnp.int32)
    src_pad = (ar * 977) % N_NODES
    dst_pad = N_NODES + ar % (N_PAD - N_NODES)
    src_flat = jnp.concatenate([src, src_pad])
    dst_flat = jnp.concatenate([dst, dst_pad])
    srcp = src_flat.reshape(E_PAD // SLAB, SLAB)
    dstp = dst_flat.reshape(E_PAD // SLAB, SLAB)
    dstc = dst_flat.reshape(E_PAD // DSLAB, DSLAB)

    deg_part = _deg_kernel(dstc)
    xs1, dis = _tc1(deg_part, x, W1)
    agg1 = _agg_mid(xs1, srcp, dstp)
    xs2h = _tc2(agg1, xs1, dis, b1.reshape(1, D_MID), W2)
    return _agg_final(xs2h, srcp, dstp, dis, b2.reshape(NC, D_MID))


# speedup vs baseline: 1.0307x; 1.0307x over previous
import functools

import jax
import jax.numpy as jnp
from jax import lax
from jax.experimental import pallas as pl
from jax.experimental.pallas import tpu as pltpu
from jax.experimental.pallas import tpu_sc as plsc

N_NODES = 10000
N_PAD = 10240
D_IN = 128
D_MID = 64
N_EDGES = 320000

NC = 2
NS = 16
NW = NC * NS
CHUNK = 128
CPT = 80
E_PAD = NW * CPT * CHUNK
RPT = N_PAD // NS

_MESH = plsc.VectorSubcoreMesh(core_axis_name="c", subcore_axis_name="s")
_SC_PARAMS = pltpu.CompilerParams(use_tc_tiling_on_sc=False)


def _fill(ref, val, n):
    v = jnp.full((16,), val, jnp.float32)
    for k in range(n // 16):
        ref[pl.ds(k * 16, 16)] = v


DSLAB = 512
DSPT = E_PAD // NW // DSLAB


@functools.partial(
    pl.kernel,
    out_type=jax.ShapeDtypeStruct((NC, N_PAD), jnp.float32),
    mesh=_MESH,
    scratch_types=[
        pltpu.VMEM((DSPT, DSLAB), jnp.int32),
        pltpu.VMEM((DSLAB,), jnp.float32),
        pltpu.VMEM((DSLAB,), jnp.float32),
        pltpu.VMEM_SHARED((N_PAD,), jnp.float32),
        pltpu.SemaphoreType.DMA,
    ],
    compiler_params=_SC_PARAMS,
)
def _deg_kernel(dst_hbm, out_hbm, dst_v, ones_v, zeros_v, acc, sem):
    c = lax.axis_index("c")
    s = lax.axis_index("s")
    wid = c * NS + s
    pltpu.sync_copy(dst_hbm.at[pl.ds(wid * DSPT, DSPT)], dst_v)
    _fill(ones_v, 1.0, DSLAB)
    _fill(zeros_v, 0.0, DSLAB)
    base = s * RPT
    off = 0
    while off < RPT:
        step = min(DSLAB, RPT - off)
        pltpu.sync_copy(zeros_v.at[pl.ds(0, step)],
                        acc.at[pl.ds(base + off, step)])
        off += step
    plsc.subcore_barrier()

    def group(i, _):
        for b in range(4):
            pltpu.async_copy(ones_v, acc.at[dst_v.at[i * 4 + b]], sem,
                             add=True)
        for b in range(4):
            pltpu.make_async_copy(ones_v, acc.at[dst_v.at[i * 4 + b]],
                                  sem).wait()
        return 0

    lax.fori_loop(0, DSPT // 4, group, 0)
    plsc.subcore_barrier()
    pltpu.sync_copy(acc.at[pl.ds(base, RPT)], out_hbm.at[c, pl.ds(base, RPT)])


SLAB = 160


def _make_agg(core_split):
    ept = (E_PAD // NS if core_split else E_PAD // NW)
    ns = ept // SLAB
    D = D_MID

    @functools.partial(
        pl.kernel,
        out_type=jax.ShapeDtypeStruct((NC, N_PAD, D), jnp.float32),
        mesh=_MESH,
        scratch_types=[
            pltpu.VMEM((ns, SLAB), jnp.int32),
            pltpu.VMEM((ns, SLAB), jnp.int32),
            pltpu.VMEM((SLAB, D), jnp.float32),
            pltpu.VMEM((SLAB, D), jnp.float32),
            pltpu.VMEM((SLAB, D), jnp.float32),
            pltpu.VMEM((SLAB, D), jnp.float32),
            pltpu.VMEM_SHARED((N_PAD, D), jnp.float32),
            pltpu.SemaphoreType.DMA,
            pltpu.SemaphoreType.DMA,
            pltpu.SemaphoreType.DMA,
            pltpu.SemaphoreType.DMA,
        ],
        compiler_params=_SC_PARAMS,
    )
    def agg(xs_hbm, src_hbm, dst_hbm, out_hbm, src_v, dst_v, rows0, rows1,
            rows2, rows3, acc, gsem0, gsem1, ssem0, ssem1):
        c = lax.axis_index("c")
        s = lax.axis_index("s")
        if core_split:
            sbase = s * ns
            xsc = xs_hbm.at[c]
        else:
            sbase = (c * NS + s) * ns
            xsc = xs_hbm
        pltpu.sync_copy(src_hbm.at[pl.ds(sbase, ns)], src_v)
        pltpu.sync_copy(dst_hbm.at[pl.ds(sbase, ns)], dst_v)

        def zrow(r, _):
            for k in range(D // 16):
                rows0[r, pl.ds(k * 16, 16)] = jnp.zeros((16,), jnp.float32)
            return 0

        lax.fori_loop(0, SLAB, zrow, 0)
        base = s * RPT
        off = 0
        while off < RPT:
            step = min(SLAB, RPT - off)
            pltpu.sync_copy(rows0.at[pl.ds(0, step)],
                            acc.at[pl.ds(base + off, step)])
            off += step
        plsc.subcore_barrier()

        def sidx(j):
            return src_v.at[j]

        def didx(j):
            return dst_v.at[j]

        rows = [rows0, rows1, rows2, rows3]
        gsem = [gsem0, gsem1]
        ssem = [ssem0, ssem1]

        pltpu.async_copy(xsc.at[sidx(0)], rows[0], gsem[0])
        pltpu.async_copy(xsc.at[sidx(1)], rows[1], gsem[1])

        def quad(i, _):
            for b in range(4):
                j = i * 4 + b
                pltpu.make_async_copy(
                    xsc.at[sidx(j)], rows[b], gsem[b % 2]).wait()
                pltpu.async_copy(rows[b], acc.at[didx(j)], ssem[b % 2],
                                 add=True)

                @pl.when(j >= 2)
                def _():
                    pltpu.make_async_copy(
                        rows[(b + 2) % 4], acc.at[didx(j - 2)],
                        ssem[b % 2]).wait()

                @pl.when(j + 2 < ns)
                def _():
                    pltpu.async_copy(
                        xsc.at[sidx(j + 2)], rows[(b + 2) % 4], gsem[b % 2])
            return 0

        lax.fori_loop(0, ns // 4, quad, 0)
        pltpu.make_async_copy(
            rows[(ns - 2) % 4], acc.at[didx(ns - 2)], ssem[0]).wait()
        pltpu.make_async_copy(
            rows[(ns - 1) % 4], acc.at[didx(ns - 1)], ssem[1]).wait()

        plsc.subcore_barrier()
        pltpu.sync_copy(acc.at[pl.ds(base, RPT)],
                        out_hbm.at[c, pl.ds(base, RPT)])

    return agg


_agg_mid = _make_agg(core_split=False)


_NS2 = (E_PAD // NS) // SLAB


@functools.partial(
    pl.kernel,
    out_type=jax.ShapeDtypeStruct((N_NODES, D_IN), jnp.float32),
    mesh=_MESH,
    scratch_types=[
        pltpu.VMEM((_NS2, SLAB), jnp.int32),
        pltpu.VMEM((_NS2, SLAB), jnp.int32),
        pltpu.VMEM((SLAB, D_MID), jnp.float32),
        pltpu.VMEM((SLAB, D_MID), jnp.float32),
        pltpu.VMEM((SLAB, D_MID), jnp.float32),
        pltpu.VMEM((SLAB, D_MID), jnp.float32),
        pltpu.VMEM((RPT,), jnp.float32),
        pltpu.VMEM((D_MID,), jnp.float32),
        pltpu.VMEM_SHARED((N_PAD, D_MID), jnp.float32),
        pltpu.SemaphoreType.DMA,
        pltpu.SemaphoreType.DMA,
        pltpu.SemaphoreType.DMA,
        pltpu.SemaphoreType.DMA,
    ],
    compiler_params=_SC_PARAMS,
)
def _agg_final(xs_hbm, src_hbm, dst_hbm, dis_hbm, b2_hbm, out_hbm,
               src_v, dst_v, rows0, rows1, rows2, rows3, dis_v, b2_v,
               acc, gsem0, gsem1, ssem0, ssem1):
    ns = _NS2
    D = D_MID
    c = lax.axis_index("c")
    s = lax.axis_index("s")
    sbase = s * ns
    xsc = xs_hbm.at[c]
    pltpu.sync_copy(src_hbm.at[pl.ds(sbase, ns)], src_v)
    pltpu.sync_copy(dst_hbm.at[pl.ds(sbase, ns)], dst_v)
    base = s * RPT
    pltpu.sync_copy(dis_hbm.at[pl.ds(base, RPT)], dis_v)
    pltpu.sync_copy(b2_hbm.at[c], b2_v)

    def zrow(r, _):
        for k in range(D // 16):
            rows0[r, pl.ds(k * 16, 16)] = jnp.zeros((16,), jnp.float32)
        return 0

    lax.fori_loop(0, SLAB, zrow, 0)
    off = 0
    while off < RPT:
        step = min(SLAB, RPT - off)
        pltpu.sync_copy(rows0.at[pl.ds(0, step)],
                        acc.at[pl.ds(base + off, step)])
        off += step
    plsc.subcore_barrier()

    rows = [rows0, rows1, rows2, rows3]
    gsem = [gsem0, gsem1]
    ssem = [ssem0, ssem1]

    pltpu.async_copy(xsc.at[src_v.at[0]], rows[0], gsem[0])
    pltpu.async_copy(xsc.at[src_v.at[1]], rows[1], gsem[1])

    def quad(i, _):
        for b in range(4):
            j = i * 4 + b
            pltpu.make_async_copy(
                xsc.at[src_v.at[j]], rows[b], gsem[b % 2]).wait()
            pltpu.async_copy(rows[b], acc.at[dst_v.at[j]], ssem[b % 2],
                             add=True)

            @pl.when(j >= 2)
            def _():
                pltpu.make_async_copy(
                    rows[(b + 2) % 4], acc.at[dst_v.at[j - 2]],
                    ssem[b % 2]).wait()

            @pl.when(j + 2 < ns)
            def _():
                pltpu.async_copy(
                    xsc.at[src_v.at[j + 2]], rows[(b + 2) % 4], gsem[b % 2])
        return 0

    lax.fori_loop(0, ns // 4, quad, 0)
    pltpu.make_async_copy(
        rows[(ns - 2) % 4], acc.at[dst_v.at[ns - 2]], ssem[0]).wait()
    pltpu.make_async_copy(
        rows[(ns - 1) % 4], acc.at[dst_v.at[ns - 1]], ssem[1]).wait()

    plsc.subcore_barrier()

    for t in range(RPT // SLAB):
        r0 = base + t * SLAB
        pltpu.sync_copy(acc.at[pl.ds(r0, SLAB)], rows0)
        pltpu.sync_copy(xsc.at[pl.ds(r0, SLAB)], rows1)

        def fgroup(g, _):
            dvec = dis_v[pl.ds(t * SLAB + g * 16, 16)]
            for rr in range(16):
                r = g * 16 + rr
                dv = dvec[rr]
                for k in range(D // 16):
                    sl = pl.ds(k * 16, 16)
                    rows0[r, sl] = ((rows0[r, sl] + rows1[r, sl]) * dv
                                    + b2_v[sl])
            return 0

        lax.fori_loop(0, SLAB // 16, fgroup, 0)
        tail = N_NODES % SLAB

        @pl.when(r0 + SLAB <= N_NODES)
        def _():
            pltpu.sync_copy(rows0,
                            out_hbm.at[pl.ds(r0, SLAB), pl.ds(c * D, D)])

        @pl.when(jnp.logical_and(r0 < N_NODES, r0 + SLAB > N_NODES))
        def _():
            pltpu.sync_copy(
                rows0.at[pl.ds(0, tail)],
                out_hbm.at[pl.ds(r0, tail), pl.ds(c * D, D)])


def _tc1_body(deg_ref, x_ref, w_ref, xs_ref, dis_ref):
    deg = deg_ref[0] + deg_ref[1] + 1.0
    dis1 = lax.rsqrt(deg)
    dis_ref[...] = dis1
    dis = dis1.reshape(N_PAD, 1)
    xw = jnp.dot(x_ref[...], w_ref[...], preferred_element_type=jnp.float32)
    xs_ref[pl.ds(0, N_NODES), :] = xw * dis[:N_NODES]
    xs_ref[pl.ds(N_NODES, N_PAD - N_NODES), :] = jnp.zeros(
        (N_PAD - N_NODES, D_MID), jnp.float32)


def _tc2_body(agg_ref, xs_ref, dis_ref, b_ref, w_ref, xs2_ref):
    dis = dis_ref[...].reshape(N_PAD, 1)
    h = jnp.maximum(
        dis * (agg_ref[0] + agg_ref[1] + xs_ref[...]) + b_ref[...], 0.0)
    xw2 = jnp.dot(h, w_ref[...], preferred_element_type=jnp.float32)
    xs2 = xw2 * dis
    xs2_ref[0] = xs2[:, :D_MID]
    xs2_ref[1] = xs2[:, D_MID:]


_tc1 = pl.pallas_call(
    _tc1_body,
    out_shape=[
        jax.ShapeDtypeStruct((N_PAD, D_MID), jnp.float32),
        jax.ShapeDtypeStruct((N_PAD,), jnp.float32),
    ],
)

_tc2 = pl.pallas_call(
    _tc2_body,
    out_shape=jax.ShapeDtypeStruct((NC, N_PAD, D_MID), jnp.float32),
)


def kernel(x, edge_index, W1, b1, W2, b2):
    src = edge_index[0].astype(jnp.int32)
    dst = edge_index[1].astype(jnp.int32)
    pad = E_PAD - N_EDGES
    ar = jnp.arange(pad, dtype=jnp.int32)
    src_pad = (ar * 977) % N_NODES
    dst_pad = N_NODES + ar % (N_PAD - N_NODES)
    src_flat = jnp.concatenate([src, src_pad])
    dst_flat = jnp.concatenate([dst, dst_pad])
    srcp = src_flat.reshape(E_PAD // SLAB, SLAB)
    dstp = dst_flat.reshape(E_PAD // SLAB, SLAB)
    dstc = dst_flat.reshape(E_PAD // DSLAB, DSLAB)

    deg_part = _deg_kernel(dstc)
    xs1, dis = _tc1(deg_part, x, W1)
    agg1 = _agg_mid(xs1, srcp, dstp)
    xs2h = _tc2(agg1, xs1, dis, b1.reshape(1, D_MID), W2)
    return _agg_final(xs2h, srcp, dstp, dis, b2.reshape(NC, D_MID))
